# Initial kernel scaffold; baseline (speedup 1.0000x reference)
import jax
import jax.numpy as jnp
from jax.experimental import pallas as pl


def kernel(feat_src, edge_index, feat_edge, W_src, W_dst, b_dst, W_attn_src, W_attn_edge):
    raise NotImplementedError("write your pallas kernel here")


# speedup vs baseline: 11.8162x; 11.8162x over previous
import functools

import jax
import jax.numpy as jnp
from jax import lax
from jax.experimental import pallas as pl
from jax.experimental.pallas import tpu as pltpu
from jax.experimental.pallas import tpu_sc as plsc

N = 10000
E = 320000
D = 128
D_EDGE = 16

NC = 2
NS = 16
NW = NC * NS
EPT = E // NW
CHUNK = 128
NJ = (EPT + CHUNK - 1) // CHUNK
EPAD = NJ * CHUNK
STRIPE = 640
NPAD = NS * STRIPE


def _matmul_body(x_ref, fe_ref, ws_ref, wd_ref, b_ref, was_ref, wae_ref,
                 fcs_ref, fcd_ref, asrc_ref, ae_ref):
    x = x_ref[...]
    fcs_ref[...] = jnp.dot(x, ws_ref[...], preferred_element_type=jnp.float32)
    fcd_ref[...] = jnp.dot(x, wd_ref[...], preferred_element_type=jnp.float32) + b_ref[...]
    asrc_ref[...] = jnp.dot(x, was_ref[...], preferred_element_type=jnp.float32)
    ae_ref[...] = jnp.dot(fe_ref[...], wae_ref[...], preferred_element_type=jnp.float32)


def _sc_body(asrc_hbm, src_hbm, dst_hbm, ae_hbm, fc_hbm,
             part_hbm, s_hbm,
             src_v, dst_v, ex_v, av_v, rows_v, zero1_v,
             acc_sh, s_sh, sem):
    cid = lax.axis_index("c")
    sid = lax.axis_index("s")
    wid = cid * NS + sid
    base = pl.multiple_of(sid * STRIPE, STRIPE)

    pltpu.sync_copy(src_hbm.at[wid], src_v)
    pltpu.sync_copy(dst_hbm.at[wid], dst_v)
    pltpu.sync_copy(ae_hbm.at[wid], ex_v)

    z16 = jnp.zeros((16,), jnp.float32)

    def zrows(i, c):
        rows_v[i // 8, pl.ds((i % 8) * 16, 16)] = z16
        return c
    lax.fori_loop(0, CHUNK * 8, zrows, 0)

    def z1(i, c):
        zero1_v[pl.ds(i * 16, 16)] = z16
        return c
    lax.fori_loop(0, STRIPE // 16, z1, 0)

    for b in range(STRIPE // CHUNK):
        pltpu.sync_copy(rows_v, acc_sh.at[pl.ds(base + b * CHUNK, CHUNK)])
    pltpu.sync_copy(zero1_v, s_sh.at[pl.ds(base, STRIPE)])

    plsc.subcore_barrier()

    def edge_body(j, c):
        pltpu.async_copy(asrc_hbm.at[src_v.at[j]], av_v, sem).wait()
        pltpu.async_copy(fc_hbm.at[src_v.at[j]], rows_v, sem).wait()

        for k in range(8):
            o = k * 16
            ex_v[j, pl.ds(o, 16)] = jnp.exp(av_v[pl.ds(o, 16)] + ex_v[j, pl.ds(o, 16)])

        def scale_body(r, c2):
            a = plsc.load_gather(ex_v.at[j], [jnp.full((16,), r, jnp.int32)])
            for k in range(8):
                rows_v[r, pl.ds(k * 16, 16)] = rows_v[r, pl.ds(k * 16, 16)] * a
            return c2
        lax.fori_loop(0, CHUNK, scale_body, 0)

        pltpu.sync_copy(ex_v.at[j], s_sh.at[dst_v.at[j]], add=True)
        pltpu.sync_copy(rows_v, acc_sh.at[dst_v.at[j]], add=True)
        return c
    lax.fori_loop(0, NJ, edge_body, 0)

    plsc.subcore_barrier()

    for b in range(STRIPE // CHUNK):
        pltpu.sync_copy(acc_sh.at[pl.ds(base + b * CHUNK, CHUNK)], rows_v)
        pltpu.sync_copy(rows_v, part_hbm.at[cid].at[pl.ds(base + b * CHUNK, CHUNK)])
    pltpu.sync_copy(s_sh.at[pl.ds(base, STRIPE)], zero1_v)
    pltpu.sync_copy(zero1_v, s_hbm.at[cid].at[pl.ds(base, STRIPE)])


def _epilogue_body(p0_ref, p1_ref, s0_ref, s1_ref, fcd_ref, out_ref):
    s = s0_ref[...] + s1_ref[...]
    r = 1.0 / (s + 1e-16)
    out_ref[...] = (p0_ref[...] + p1_ref[...]) * r + fcd_ref[...]


@jax.jit
def kernel(feat_src, edge_index, feat_edge, W_src, W_dst, b_dst, W_attn_src, W_attn_edge):
    src = edge_index[0]
    dst = edge_index[1]

    was_p = jnp.pad(W_attn_src, ((0, 0), (0, 7)))
    wae_p = jnp.pad(W_attn_edge, ((0, 0), (0, 7)))
    b2 = b_dst.reshape(1, D)

    g = 25
    bn = N // g
    be = E // g
    fc_src, fc_dst, asrc8, ae8 = pl.pallas_call(
        _matmul_body,
        grid=(g,),
        in_specs=[
            pl.BlockSpec((bn, D), lambda i: (i, 0)),
            pl.BlockSpec((be, D_EDGE), lambda i: (i, 0)),
            pl.BlockSpec((D, D), lambda i: (0, 0)),
            pl.BlockSpec((D, D), lambda i: (0, 0)),
            pl.BlockSpec((1, D), lambda i: (0, 0)),
            pl.BlockSpec((D, 8), lambda i: (0, 0)),
            pl.BlockSpec((D_EDGE, 8), lambda i: (0, 0)),
        ],
        out_specs=[
            pl.BlockSpec((bn, D), lambda i: (i, 0)),
            pl.BlockSpec((bn, D), lambda i: (i, 0)),
            pl.BlockSpec((bn, 8), lambda i: (i, 0)),
            pl.BlockSpec((be, 8), lambda i: (i, 0)),
        ],
        out_shape=[
            jax.ShapeDtypeStruct((N, D), jnp.float32),
            jax.ShapeDtypeStruct((N, D), jnp.float32),
            jax.ShapeDtypeStruct((N, 8), jnp.float32),
            jax.ShapeDtypeStruct((E, 8), jnp.float32),
        ],
    )(feat_src, feat_edge, W_src, W_dst, b2, was_p, wae_p)

    asrc = asrc8[:, 0]
    aedge = ae8[:, 0]

    pad = EPAD - EPT
    src_p = jnp.pad(src.reshape(NW, EPT), ((0, 0), (0, pad))).reshape(NW, NJ, CHUNK)
    dst_p = jnp.pad(dst.reshape(NW, EPT), ((0, 0), (0, pad)),
                    constant_values=N).reshape(NW, NJ, CHUNK)
    ae_p = jnp.pad(aedge.reshape(NW, EPT), ((0, 0), (0, pad)),
                   constant_values=-1e30).reshape(NW, NJ, CHUNK)

    sc_fn = pl.kernel(
        _sc_body,
        out_type=(
            jax.ShapeDtypeStruct((NC, NPAD, D), jnp.float32),
            jax.ShapeDtypeStruct((NC, NPAD), jnp.float32),
        ),
        mesh=plsc.VectorSubcoreMesh(core_axis_name="c", subcore_axis_name="s"),
        compiler_params=pltpu.CompilerParams(needs_layout_passes=False),
        scratch_types=[
            pltpu.VMEM((NJ, CHUNK), jnp.int32),
            pltpu.VMEM((NJ, CHUNK), jnp.int32),
            pltpu.VMEM((NJ, CHUNK), jnp.float32),
            pltpu.VMEM((CHUNK,), jnp.float32),
            pltpu.VMEM((CHUNK, D), jnp.float32),
            pltpu.VMEM((STRIPE,), jnp.float32),
            pltpu.VMEM_SHARED((NPAD, D), jnp.float32),
            pltpu.VMEM_SHARED((NPAD,), jnp.float32),
            pltpu.SemaphoreType.DMA,
        ],
    )
    part, s_part = sc_fn(asrc, src_p, dst_p, ae_p, fc_src)

    ge = 10
    bo = N // ge
    out = pl.pallas_call(
        _epilogue_body,
        grid=(ge,),
        in_specs=[
            pl.BlockSpec((bo, D), lambda i: (i, 0)),
            pl.BlockSpec((bo, D), lambda i: (i, 0)),
            pl.BlockSpec((bo, 1), lambda i: (i, 0)),
            pl.BlockSpec((bo, 1), lambda i: (i, 0)),
            pl.BlockSpec((bo, D), lambda i: (i, 0)),
        ],
        out_specs=pl.BlockSpec((bo, D), lambda i: (i, 0)),
        out_shape=jax.ShapeDtypeStruct((N, D), jnp.float32),
    )(part[0, :N], part[1, :N],
      s_part[0, :N].reshape(N, 1), s_part[1, :N].reshape(N, 1), fc_dst)

    return out.reshape(N, 1, D)


# speedup vs baseline: 13.8941x; 1.1759x over previous
import functools

import jax
import jax.numpy as jnp
from jax import lax
from jax.experimental import pallas as pl
from jax.experimental.pallas import tpu as pltpu
from jax.experimental.pallas import tpu_sc as plsc

N = 10000
E = 320000
D = 128
D_EDGE = 16

NC = 2
NS = 16
NW = NC * NS
EPT = E // NW
CHUNK = 128
NJ = (EPT + CHUNK - 1) // CHUNK
EPAD = NJ * CHUNK
STRIPE = 640
NPAD = NS * STRIPE


def _matmul_body(x_ref, fe_ref, ws_ref, wd_ref, b_ref, was_ref, wae_ref,
                 fcs_ref, fcd_ref, asrc_ref, ae_ref):
    x = x_ref[...]
    fcs_ref[...] = jnp.dot(x, ws_ref[...], preferred_element_type=jnp.float32)
    fcd_ref[...] = jnp.dot(x, wd_ref[...], preferred_element_type=jnp.float32) + b_ref[...]
    asrc_ref[...] = jnp.dot(x, was_ref[...], preferred_element_type=jnp.float32)
    ae_ref[...] = jnp.dot(fe_ref[...], wae_ref[...], preferred_element_type=jnp.float32)


def _sc_body(asrc_hbm, edata_hbm, fc_hbm,
             part_hbm, s_hbm,
             eb, avb, exb, rows_v, zero1_v,
             acc_sh, s_sh, sem):
    cid = lax.axis_index("c")
    sid = lax.axis_index("s")
    wid = cid * NS + sid
    base = pl.multiple_of(sid * STRIPE, STRIPE)

    z16 = jnp.zeros((16,), jnp.float32)

    def zrows(i, c):
        rows_v[0, i // 8, pl.ds((i % 8) * 16, 16)] = z16
        return c
    lax.fori_loop(0, CHUNK * 8, zrows, 0)

    def z1(i, c):
        zero1_v[pl.ds(i * 16, 16)] = z16
        return c
    lax.fori_loop(0, STRIPE // 16, z1, 0)

    for b in range(STRIPE // CHUNK):
        pltpu.sync_copy(rows_v.at[0], acc_sh.at[pl.ds(base + b * CHUNK, CHUNK)])
    pltpu.sync_copy(zero1_v, s_sh.at[pl.ds(base, STRIPE)])

    plsc.subcore_barrier()

    pltpu.sync_copy(edata_hbm.at[wid].at[0], eb.at[pl.ds(0, 3)])
    pltpu.async_copy(asrc_hbm.at[eb.at[0]], avb.at[0], sem)
    pltpu.async_copy(fc_hbm.at[eb.at[0]], rows_v.at[0], sem)

    def edge_body(j, c):
        b = lax.rem(j, 2)
        nb = 1 - b
        e0 = b * 3
        ne0 = nb * 3
        pltpu.make_async_copy(asrc_hbm.at[eb.at[e0]], avb.at[b], sem).wait()
        pltpu.make_async_copy(fc_hbm.at[eb.at[e0]], rows_v.at[b], sem).wait()

        @pl.when(j + 1 < NJ)
        def _prefetch():
            pltpu.sync_copy(edata_hbm.at[wid].at[j + 1], eb.at[pl.ds(ne0, 3)])
            pltpu.async_copy(asrc_hbm.at[eb.at[ne0]], avb.at[nb], sem)
            pltpu.async_copy(fc_hbm.at[eb.at[ne0]], rows_v.at[nb], sem)

        for k in range(8):
            o = k * 16
            ae = plsc.bitcast(eb[e0 + 2, pl.ds(o, 16)], jnp.float32)
            exb[b, pl.ds(o, 16)] = jnp.exp(avb[b, pl.ds(o, 16)] + ae)

        def scale_body(r, c2):
            a = plsc.load_gather(exb.at[b], [jnp.full((16,), r, jnp.int32)])
            for k in range(8):
                rows_v[b, r, pl.ds(k * 16, 16)] = rows_v[b, r, pl.ds(k * 16, 16)] * a
            return c2
        lax.fori_loop(0, CHUNK, scale_body, 0)

        pltpu.sync_copy(exb.at[b], s_sh.at[eb.at[e0 + 1]], add=True)
        pltpu.sync_copy(rows_v.at[b], acc_sh.at[eb.at[e0 + 1]], add=True)
        return c
    lax.fori_loop(0, NJ, edge_body, 0)

    plsc.subcore_barrier()

    for b in range(STRIPE // CHUNK):
        pltpu.sync_copy(acc_sh.at[pl.ds(base + b * CHUNK, CHUNK)], rows_v.at[0])
        pltpu.sync_copy(rows_v.at[0], part_hbm.at[cid].at[pl.ds(base + b * CHUNK, CHUNK)])
    pltpu.sync_copy(s_sh.at[pl.ds(base, STRIPE)], zero1_v)
    pltpu.sync_copy(zero1_v, s_hbm.at[cid].at[pl.ds(base, STRIPE)])


def _epilogue_body(p_ref, s0_ref, s1_ref, fcd_ref, out_ref):
    p = p_ref[...]
    s = s0_ref[...] + s1_ref[...]
    r = 1.0 / (s + 1e-16)
    out_ref[...] = (p[0] + p[1]) * r + fcd_ref[...]


@jax.jit
def kernel(feat_src, edge_index, feat_edge, W_src, W_dst, b_dst, W_attn_src, W_attn_edge):
    src = edge_index[0]
    dst = edge_index[1]

    was_p = jnp.pad(W_attn_src, ((0, 0), (0, 7)))
    wae_p = jnp.pad(W_attn_edge, ((0, 0), (0, 7)))
    b2 = b_dst.reshape(1, D)

    g = 25
    bn = N // g
    be = E // g
    fc_src, fc_dst, asrc8, ae8 = pl.pallas_call(
        _matmul_body,
        grid=(g,),
        in_specs=[
            pl.BlockSpec((bn, D), lambda i: (i, 0)),
            pl.BlockSpec((be, D_EDGE), lambda i: (i, 0)),
            pl.BlockSpec((D, D), lambda i: (0, 0)),
            pl.BlockSpec((D, D), lambda i: (0, 0)),
            pl.BlockSpec((1, D), lambda i: (0, 0)),
            pl.BlockSpec((D, 8), lambda i: (0, 0)),
            pl.BlockSpec((D_EDGE, 8), lambda i: (0, 0)),
        ],
        out_specs=[
            pl.BlockSpec((bn, D), lambda i: (i, 0)),
            pl.BlockSpec((bn, D), lambda i: (i, 0)),
            pl.BlockSpec((bn, 8), lambda i: (i, 0)),
            pl.BlockSpec((be, 8), lambda i: (i, 0)),
        ],
        out_shape=[
            jax.ShapeDtypeStruct((N, D), jnp.float32),
            jax.ShapeDtypeStruct((N, D), jnp.float32),
            jax.ShapeDtypeStruct((N, 8), jnp.float32),
            jax.ShapeDtypeStruct((E, 8), jnp.float32),
        ],
    )(feat_src, feat_edge, W_src, W_dst, b2, was_p, wae_p)

    asrc = asrc8[:, 0]
    aedge = ae8[:, 0]

    pad = EPAD - EPT
    src_p = jnp.pad(src.reshape(NW, EPT), ((0, 0), (0, pad))).reshape(NW, NJ, CHUNK)
    dst_p = jnp.pad(dst.reshape(NW, EPT), ((0, 0), (0, pad)),
                    constant_values=N).reshape(NW, NJ, CHUNK)
    ae_p = jnp.pad(aedge.reshape(NW, EPT), ((0, 0), (0, pad)),
                   constant_values=-1e30).reshape(NW, NJ, CHUNK)
    edata = jnp.stack(
        [src_p, dst_p, lax.bitcast_convert_type(ae_p, jnp.int32)], axis=2)

    sc_fn = pl.kernel(
        _sc_body,
        out_type=(
            jax.ShapeDtypeStruct((NC, NPAD, D), jnp.float32),
            jax.ShapeDtypeStruct((NC, NPAD), jnp.float32),
        ),
        mesh=plsc.VectorSubcoreMesh(core_axis_name="c", subcore_axis_name="s"),
        compiler_params=pltpu.CompilerParams(needs_layout_passes=False),
        scratch_types=[
            pltpu.VMEM((6, CHUNK), jnp.int32),
            pltpu.VMEM((2, CHUNK), jnp.float32),
            pltpu.VMEM((2, CHUNK), jnp.float32),
            pltpu.VMEM((2, CHUNK, D), jnp.float32),
            pltpu.VMEM((STRIPE,), jnp.float32),
            pltpu.VMEM_SHARED((NPAD, D), jnp.float32),
            pltpu.VMEM_SHARED((NPAD,), jnp.float32),
            pltpu.SemaphoreType.DMA,
        ],
    )
    part, s_part = sc_fn(asrc, edata, fc_src)

    ge = 10
    bo = N // ge
    out = pl.pallas_call(
        _epilogue_body,
        grid=(ge,),
        in_specs=[
            pl.BlockSpec((NC, bo, D), lambda i: (0, i, 0)),
            pl.BlockSpec((bo, 1), lambda i: (i, 0)),
            pl.BlockSpec((bo, 1), lambda i: (i, 0)),
            pl.BlockSpec((bo, D), lambda i: (i, 0)),
        ],
        out_specs=pl.BlockSpec((bo, D), lambda i: (i, 0)),
        out_shape=jax.ShapeDtypeStruct((N, D), jnp.float32),
    )(part,
      s_part[0, :N].reshape(N, 1), s_part[1, :N].reshape(N, 1), fc_dst)

    return out.reshape(N, 1, D)


# speedup vs baseline: 19.6682x; 1.4156x over previous
import functools

import jax
import jax.numpy as jnp
from jax import lax
from jax.experimental import pallas as pl
from jax.experimental.pallas import tpu as pltpu
from jax.experimental.pallas import tpu_sc as plsc

N = 10000
E = 320000
D = 128
D_EDGE = 16

NC = 2
NS = 16
NW = NC * NS
EPT = E // NW
CHUNK = 128
NJ = (EPT + CHUNK - 1) // CHUNK
EPAD = NJ * CHUNK
STRIPE = 640
NPAD = NS * STRIPE


def _matmul_body(x_ref, fet_ref, ws_ref, wd_ref, b_ref, was_ref, waet_ref,
                 fcs_ref, fcd_ref, asrc_ref, aet_ref):
    x = x_ref[...]
    fcs_ref[...] = jnp.dot(x, ws_ref[...], preferred_element_type=jnp.float32)
    fcd_ref[...] = jnp.dot(x, wd_ref[...], preferred_element_type=jnp.float32) + b_ref[...]
    asrc_ref[...] = jnp.dot(x, was_ref[...], preferred_element_type=jnp.float32)
    aet_ref[...] = jnp.dot(waet_ref[...], fet_ref[...], preferred_element_type=jnp.float32)


def _sc_body(asrc_hbm, edata_hbm, fc_hbm,
             part_hbm, s_hbm,
             eb, avb, exb, rows_v, zero1_v,
             acc_sh, s_sh, sem):
    cid = lax.axis_index("c")
    sid = lax.axis_index("s")
    wid = cid * NS + sid
    base = pl.multiple_of(sid * STRIPE, STRIPE)

    z16 = jnp.zeros((16,), jnp.float32)

    def zrows(i, c):
        rows_v[0, i // 8, pl.ds((i % 8) * 16, 16)] = z16
        return c
    lax.fori_loop(0, CHUNK * 8, zrows, 0)

    def z1(i, c):
        zero1_v[pl.ds(i * 16, 16)] = z16
        return c
    lax.fori_loop(0, STRIPE // 16, z1, 0)

    for b in range(STRIPE // CHUNK):
        pltpu.sync_copy(rows_v.at[0], acc_sh.at[pl.ds(base + b * CHUNK, CHUNK)])
    pltpu.sync_copy(zero1_v, s_sh.at[pl.ds(base, STRIPE)])

    plsc.subcore_barrier()

    pltpu.sync_copy(edata_hbm.at[wid].at[0], eb.at[pl.ds(0, 3)])
    pltpu.async_copy(asrc_hbm.at[eb.at[0]], avb.at[0], sem)
    pltpu.async_copy(fc_hbm.at[eb.at[0]], rows_v.at[0], sem)

    def edge_body(j, c):
        b = lax.rem(j, 2)
        nb = 1 - b
        e0 = b * 3
        ne0 = nb * 3
        pltpu.make_async_copy(asrc_hbm.at[eb.at[e0]], avb.at[b], sem).wait()
        pltpu.make_async_copy(fc_hbm.at[eb.at[e0]], rows_v.at[b], sem).wait()

        @pl.when(j + 1 < NJ)
        def _prefetch():
            pltpu.sync_copy(edata_hbm.at[wid].at[j + 1], eb.at[pl.ds(ne0, 3)])
            pltpu.async_copy(asrc_hbm.at[eb.at[ne0]], avb.at[nb], sem)
            pltpu.async_copy(fc_hbm.at[eb.at[ne0]], rows_v.at[nb], sem)

        for k in range(8):
            o = k * 16
            ae = plsc.bitcast(eb[e0 + 2, pl.ds(o, 16)], jnp.float32)
            exb[b, pl.ds(o, 16)] = jnp.exp(avb[b, pl.ds(o, 16)] + ae)

        def scale_body(r, c2):
            a = plsc.load_gather(exb.at[b], [jnp.full((16,), r, jnp.int32)])
            for k in range(8):
                rows_v[b, r, pl.ds(k * 16, 16)] = rows_v[b, r, pl.ds(k * 16, 16)] * a
            return c2
        lax.fori_loop(0, CHUNK, scale_body, 0)

        pltpu.sync_copy(exb.at[b], s_sh.at[eb.at[e0 + 1]], add=True)
        pltpu.sync_copy(rows_v.at[b], acc_sh.at[eb.at[e0 + 1]], add=True)
        return c
    lax.fori_loop(0, NJ, edge_body, 0)

    plsc.subcore_barrier()

    for b in range(STRIPE // CHUNK):
        pltpu.sync_copy(acc_sh.at[pl.ds(base + b * CHUNK, CHUNK)], rows_v.at[0])
        pltpu.sync_copy(rows_v.at[0], part_hbm.at[cid].at[pl.ds(base + b * CHUNK, CHUNK)])
    pltpu.sync_copy(s_sh.at[pl.ds(base, STRIPE)], zero1_v)
    pltpu.sync_copy(zero1_v, s_hbm.at[cid].at[pl.ds(base, STRIPE)])


def _epilogue_body(p_ref, s0_ref, s1_ref, fcd_ref, out_ref):
    p = p_ref[...]
    s = s0_ref[...] + s1_ref[...]
    r = 1.0 / (s + 1e-16)
    out_ref[...] = (p[0] + p[1]) * r + fcd_ref[...]


@jax.jit
def kernel(feat_src, edge_index, feat_edge, W_src, W_dst, b_dst, W_attn_src, W_attn_edge):
    src = edge_index[0]
    dst = edge_index[1]

    was_p = jnp.pad(W_attn_src, ((0, 0), (0, 7)))
    waet_p = jnp.pad(W_attn_edge.T, ((0, 7), (0, 0)))
    b2 = b_dst.reshape(1, D)
    fe_t = feat_edge.T

    g = 25
    bn = N // g
    be = E // g
    fc_src, fc_dst, asrc8, ae8t = pl.pallas_call(
        _matmul_body,
        grid=(g,),
        in_specs=[
            pl.BlockSpec((bn, D), lambda i: (i, 0)),
            pl.BlockSpec((D_EDGE, be), lambda i: (0, i)),
            pl.BlockSpec((D, D), lambda i: (0, 0)),
            pl.BlockSpec((D, D), lambda i: (0, 0)),
            pl.BlockSpec((1, D), lambda i: (0, 0)),
            pl.BlockSpec((D, 8), lambda i: (0, 0)),
            pl.BlockSpec((8, D_EDGE), lambda i: (0, 0)),
        ],
        out_specs=[
            pl.BlockSpec((bn, D), lambda i: (i, 0)),
            pl.BlockSpec((bn, D), lambda i: (i, 0)),
            pl.BlockSpec((bn, 8), lambda i: (i, 0)),
            pl.BlockSpec((8, be), lambda i: (0, i)),
        ],
        out_shape=[
            jax.ShapeDtypeStruct((N, D), jnp.float32),
            jax.ShapeDtypeStruct((N, D), jnp.float32),
            jax.ShapeDtypeStruct((N, 8), jnp.float32),
            jax.ShapeDtypeStruct((8, E), jnp.float32),
        ],
    )(feat_src, fe_t, W_src, W_dst, b2, was_p, waet_p)

    asrc = asrc8[:, 0]
    aedge = ae8t[0]

    pad = EPAD - EPT
    src_p = jnp.pad(src.reshape(NW, EPT), ((0, 0), (0, pad))).reshape(NW, NJ, CHUNK)
    dst_p = jnp.pad(dst.reshape(NW, EPT), ((0, 0), (0, pad)),
                    constant_values=N).reshape(NW, NJ, CHUNK)
    ae_p = jnp.pad(aedge.reshape(NW, EPT), ((0, 0), (0, pad)),
                   constant_values=-1e30).reshape(NW, NJ, CHUNK)
    edata = jnp.stack(
        [src_p, dst_p, lax.bitcast_convert_type(ae_p, jnp.int32)], axis=2)

    sc_fn = pl.kernel(
        _sc_body,
        out_type=(
            jax.ShapeDtypeStruct((NC, NPAD, D), jnp.float32),
            jax.ShapeDtypeStruct((NC, NPAD), jnp.float32),
        ),
        mesh=plsc.VectorSubcoreMesh(core_axis_name="c", subcore_axis_name="s"),
        compiler_params=pltpu.CompilerParams(needs_layout_passes=False),
        scratch_types=[
            pltpu.VMEM((6, CHUNK), jnp.int32),
            pltpu.VMEM((2, CHUNK), jnp.float32),
            pltpu.VMEM((2, CHUNK), jnp.float32),
            pltpu.VMEM((2, CHUNK, D), jnp.float32),
            pltpu.VMEM((STRIPE,), jnp.float32),
            pltpu.VMEM_SHARED((NPAD, D), jnp.float32),
            pltpu.VMEM_SHARED((NPAD,), jnp.float32),
            pltpu.SemaphoreType.DMA,
        ],
    )
    part, s_part = sc_fn(asrc, edata, fc_src)

    ge = 10
    bo = N // ge
    out = pl.pallas_call(
        _epilogue_body,
        grid=(ge,),
        in_specs=[
            pl.BlockSpec((NC, bo, D), lambda i: (0, i, 0)),
            pl.BlockSpec((bo, 1), lambda i: (i, 0)),
            pl.BlockSpec((bo, 1), lambda i: (i, 0)),
            pl.BlockSpec((bo, D), lambda i: (i, 0)),
        ],
        out_specs=pl.BlockSpec((bo, D), lambda i: (i, 0)),
        out_shape=jax.ShapeDtypeStruct((N, D), jnp.float32),
    )(part,
      s_part[0, :N].reshape(N, 1), s_part[1, :N].reshape(N, 1), fc_dst)

    return out.reshape(N, 1, D)


# speedup vs baseline: 20.0169x; 1.0177x over previous
import functools

import jax
import jax.numpy as jnp
from jax import lax
from jax.experimental import pallas as pl
from jax.experimental.pallas import tpu as pltpu
from jax.experimental.pallas import tpu_sc as plsc

N = 10000
E = 320000
D = 128
D_EDGE = 16

NC = 2
NS = 16
NW = NC * NS
EPT = E // NW
CHUNK = 128
NJ = (EPT + CHUNK - 1) // CHUNK
EPAD = NJ * CHUNK
STRIPE = 640
NPAD = NS * STRIPE


def _matmul_body(x_ref, fet_ref, ws_ref, wd_ref, b_ref, was_ref, waet_ref,
                 fcs_ref, fcd_ref, asrc_ref, aet_ref):
    x = x_ref[...]
    fcs_ref[...] = jnp.dot(x, ws_ref[...], preferred_element_type=jnp.float32)
    fcd_ref[...] = jnp.dot(x, wd_ref[...], preferred_element_type=jnp.float32) + b_ref[...]
    asrc_ref[...] = jnp.dot(x, was_ref[...], preferred_element_type=jnp.float32)
    aet_ref[...] = jnp.dot(waet_ref[...], fet_ref[...], preferred_element_type=jnp.float32)


def _sc_body(asrc_hbm, edata_hbm, fc_hbm,
             part_hbm, s_hbm,
             eb, avb, exb, rows_v, zero1_v,
             acc_sh, s_sh, sem, semw):
    cid = lax.axis_index("c")
    sid = lax.axis_index("s")
    wid = cid * NS + sid
    base = pl.multiple_of(sid * STRIPE, STRIPE)

    z16 = jnp.zeros((16,), jnp.float32)

    def zrows(i, c):
        rows_v[0, i // 8, pl.ds((i % 8) * 16, 16)] = z16
        return c
    lax.fori_loop(0, CHUNK * 8, zrows, 0)

    def z1(i, c):
        zero1_v[pl.ds(i * 16, 16)] = z16
        return c
    lax.fori_loop(0, STRIPE // 16, z1, 0)

    for b in range(STRIPE // CHUNK):
        pltpu.sync_copy(rows_v.at[0], acc_sh.at[pl.ds(base + b * CHUNK, CHUNK)])
    pltpu.sync_copy(zero1_v, s_sh.at[pl.ds(base, STRIPE)])

    plsc.subcore_barrier()

    pltpu.sync_copy(edata_hbm.at[wid].at[0], eb.at[pl.ds(0, 3)])
    pltpu.async_copy(asrc_hbm.at[eb.at[0]], avb.at[0], sem)
    pltpu.async_copy(fc_hbm.at[eb.at[0]], rows_v.at[0], sem)

    def edge_body(j, c):
        b = lax.rem(j, 2)
        nb = 1 - b
        e0 = b * 3
        ne0 = nb * 3
        pltpu.make_async_copy(asrc_hbm.at[eb.at[e0]], avb.at[b], sem).wait()
        pltpu.make_async_copy(fc_hbm.at[eb.at[e0]], rows_v.at[b], sem).wait()

        @pl.when(j >= 1)
        def _drain():
            pltpu.make_async_copy(
                rows_v.at[nb], acc_sh.at[eb.at[ne0 + 1]], semw).wait()

        @pl.when(j + 1 < NJ)
        def _prefetch():
            pltpu.sync_copy(edata_hbm.at[wid].at[j + 1], eb.at[pl.ds(ne0, 3)])
            pltpu.async_copy(asrc_hbm.at[eb.at[ne0]], avb.at[nb], sem)
            pltpu.async_copy(fc_hbm.at[eb.at[ne0]], rows_v.at[nb], sem)

        for k in range(8):
            o = k * 16
            ae = plsc.bitcast(eb[e0 + 2, pl.ds(o, 16)], jnp.float32)
            exb[b, pl.ds(o, 16)] = jnp.exp(avb[b, pl.ds(o, 16)] + ae)

        def scale_body(r2, c2):
            for u in range(2):
                r = r2 * 2 + u
                a = plsc.load_gather(exb.at[b], [jnp.full((16,), r, jnp.int32)])
                for k in range(8):
                    rows_v[b, r, pl.ds(k * 16, 16)] = (
                        rows_v[b, r, pl.ds(k * 16, 16)] * a)
            return c2
        lax.fori_loop(0, CHUNK // 2, scale_body, 0)

        pltpu.sync_copy(exb.at[b], s_sh.at[eb.at[e0 + 1]], add=True)
        pltpu.async_copy(rows_v.at[b], acc_sh.at[eb.at[e0 + 1]], semw, add=True)
        return c
    lax.fori_loop(0, NJ, edge_body, 0)

    bl = (NJ - 1) % 2
    pltpu.make_async_copy(
        rows_v.at[bl], acc_sh.at[eb.at[bl * 3 + 1]], semw).wait()

    plsc.subcore_barrier()

    for b in range(STRIPE // CHUNK):
        pltpu.sync_copy(acc_sh.at[pl.ds(base + b * CHUNK, CHUNK)], rows_v.at[0])
        pltpu.sync_copy(rows_v.at[0], part_hbm.at[cid].at[pl.ds(base + b * CHUNK, CHUNK)])
    pltpu.sync_copy(s_sh.at[pl.ds(base, STRIPE)], zero1_v)
    pltpu.sync_copy(zero1_v, s_hbm.at[cid].at[pl.ds(base, STRIPE)])


def _epilogue_body(p_ref, s0_ref, s1_ref, fcd_ref, out_ref):
    p = p_ref[...]
    s = s0_ref[...] + s1_ref[...]
    r = 1.0 / (s + 1e-16)
    out_ref[...] = (p[0] + p[1]) * r + fcd_ref[...]


@jax.jit
def kernel(feat_src, edge_index, feat_edge, W_src, W_dst, b_dst, W_attn_src, W_attn_edge):
    src = edge_index[0]
    dst = edge_index[1]

    was_p = jnp.pad(W_attn_src, ((0, 0), (0, 7)))
    waet_p = jnp.pad(W_attn_edge.T, ((0, 7), (0, 0)))
    b2 = b_dst.reshape(1, D)
    fe_t = feat_edge.T

    g = 25
    bn = N // g
    be = E // g
    fc_src, fc_dst, asrc8, ae8t = pl.pallas_call(
        _matmul_body,
        grid=(g,),
        in_specs=[
            pl.BlockSpec((bn, D), lambda i: (i, 0)),
            pl.BlockSpec((D_EDGE, be), lambda i: (0, i)),
            pl.BlockSpec((D, D), lambda i: (0, 0)),
            pl.BlockSpec((D, D), lambda i: (0, 0)),
            pl.BlockSpec((1, D), lambda i: (0, 0)),
            pl.BlockSpec((D, 8), lambda i: (0, 0)),
            pl.BlockSpec((8, D_EDGE), lambda i: (0, 0)),
        ],
        out_specs=[
            pl.BlockSpec((bn, D), lambda i: (i, 0)),
            pl.BlockSpec((bn, D), lambda i: (i, 0)),
            pl.BlockSpec((bn, 8), lambda i: (i, 0)),
            pl.BlockSpec((8, be), lambda i: (0, i)),
        ],
        out_shape=[
            jax.ShapeDtypeStruct((N, D), jnp.float32),
            jax.ShapeDtypeStruct((N, D), jnp.float32),
            jax.ShapeDtypeStruct((N, 8), jnp.float32),
            jax.ShapeDtypeStruct((8, E), jnp.float32),
        ],
    )(feat_src, fe_t, W_src, W_dst, b2, was_p, waet_p)

    asrc = asrc8[:, 0]
    aedge = ae8t[0]

    pad = EPAD - EPT
    src_p = jnp.pad(src.reshape(NW, EPT), ((0, 0), (0, pad))).reshape(NW, NJ, CHUNK)
    dst_p = jnp.pad(dst.reshape(NW, EPT), ((0, 0), (0, pad)),
                    constant_values=N).reshape(NW, NJ, CHUNK)
    ae_p = jnp.pad(aedge.reshape(NW, EPT), ((0, 0), (0, pad)),
                   constant_values=-1e30).reshape(NW, NJ, CHUNK)
    edata = jnp.stack(
        [src_p, dst_p, lax.bitcast_convert_type(ae_p, jnp.int32)], axis=2)

    sc_fn = pl.kernel(
        _sc_body,
        out_type=(
            jax.ShapeDtypeStruct((NC, NPAD, D), jnp.float32),
            jax.ShapeDtypeStruct((NC, NPAD), jnp.float32),
        ),
        mesh=plsc.VectorSubcoreMesh(core_axis_name="c", subcore_axis_name="s"),
        compiler_params=pltpu.CompilerParams(needs_layout_passes=False),
        scratch_types=[
            pltpu.VMEM((6, CHUNK), jnp.int32),
            pltpu.VMEM((2, CHUNK), jnp.float32),
            pltpu.VMEM((2, CHUNK), jnp.float32),
            pltpu.VMEM((2, CHUNK, D), jnp.float32),
            pltpu.VMEM((STRIPE,), jnp.float32),
            pltpu.VMEM_SHARED((NPAD, D), jnp.float32),
            pltpu.VMEM_SHARED((NPAD,), jnp.float32),
            pltpu.SemaphoreType.DMA,
            pltpu.SemaphoreType.DMA,
        ],
    )
    part, s_part = sc_fn(asrc, edata, fc_src)

    ge = 10
    bo = N // ge
    out = pl.pallas_call(
        _epilogue_body,
        grid=(ge,),
        in_specs=[
            pl.BlockSpec((NC, bo, D), lambda i: (0, i, 0)),
            pl.BlockSpec((bo, 1), lambda i: (i, 0)),
            pl.BlockSpec((bo, 1), lambda i: (i, 0)),
            pl.BlockSpec((bo, D), lambda i: (i, 0)),
        ],
        out_specs=pl.BlockSpec((bo, D), lambda i: (i, 0)),
        out_shape=jax.ShapeDtypeStruct((N, D), jnp.float32),
    )(part,
      s_part[0, :N].reshape(N, 1), s_part[1, :N].reshape(N, 1), fc_dst)

    return out.reshape(N, 1, D)


# speedup vs baseline: 22.0882x; 1.1035x over previous
import functools

import jax
import jax.numpy as jnp
from jax import lax
from jax.experimental import pallas as pl
from jax.experimental.pallas import tpu as pltpu
from jax.experimental.pallas import tpu_sc as plsc

N = 10000
E = 320000
D = 128
D_EDGE = 16

NC = 2
NS = 16
NW = NC * NS
EPT = E // NW
CHUNK = 128
NJ = (EPT + CHUNK - 1) // CHUNK
EPAD = NJ * CHUNK
STRIPE = 640
NPAD = NS * STRIPE


def _matmul_body(x_ref, fet_ref, ws_ref, wd_ref, b_ref, was_ref, waet_ref,
                 fcs_ref, fcd_ref, asrc_ref, aet_ref):
    x = x_ref[...]
    fcs_ref[...] = jnp.dot(x, ws_ref[...], preferred_element_type=jnp.float32)
    fcd_ref[...] = jnp.dot(x, wd_ref[...], preferred_element_type=jnp.float32) + b_ref[...]
    asrc_ref[...] = jnp.dot(x, was_ref[...], preferred_element_type=jnp.float32)
    aet_ref[...] = jnp.dot(waet_ref[...], fet_ref[...], preferred_element_type=jnp.float32)


def _sc_body(asrc_hbm, edata_hbm, fc_hbm,
             part_hbm, s_hbm,
             eb, avb, exb, rows_v, zero1_v,
             acc_sh, s_sh, sem, semw, semi):
    cid = lax.axis_index("c")
    sid = lax.axis_index("s")
    wid = cid * NS + sid
    base = pl.multiple_of(sid * STRIPE, STRIPE)

    z16 = jnp.zeros((16,), jnp.float32)

    def zrows(i, c):
        rows_v[0, i // 8, pl.ds((i % 8) * 16, 16)] = z16
        return c
    lax.fori_loop(0, CHUNK * 8, zrows, 0)

    def z1(i, c):
        zero1_v[pl.ds(i * 16, 16)] = z16
        return c
    lax.fori_loop(0, STRIPE // 16, z1, 0)

    for b in range(STRIPE // CHUNK):
        pltpu.sync_copy(rows_v.at[0], acc_sh.at[pl.ds(base + b * CHUNK, CHUNK)])
    pltpu.sync_copy(zero1_v, s_sh.at[pl.ds(base, STRIPE)])

    plsc.subcore_barrier()

    pltpu.sync_copy(edata_hbm.at[wid].at[0], eb.at[pl.ds(0, 3)])
    pltpu.async_copy(asrc_hbm.at[eb.at[0]], avb.at[0], sem)
    pltpu.async_copy(fc_hbm.at[eb.at[0]], rows_v.at[0], sem)
    pltpu.async_copy(edata_hbm.at[wid].at[1], eb.at[pl.ds(3, 3)], semi)

    def edge_body(j, c):
        b = lax.rem(j, 2)
        nb = 1 - b
        e0 = lax.rem(j, 4) * 3
        pe0 = lax.rem(j + 3, 4) * 3
        ne0 = lax.rem(j + 1, 4) * 3
        fe0 = lax.rem(j + 2, 4) * 3
        pltpu.make_async_copy(asrc_hbm.at[eb.at[e0]], avb.at[b], sem).wait()
        pltpu.make_async_copy(fc_hbm.at[eb.at[e0]], rows_v.at[b], sem).wait()

        @pl.when(j >= 1)
        def _drain():
            pltpu.make_async_copy(
                exb.at[nb], s_sh.at[eb.at[pe0 + 1]], semw).wait()
            pltpu.make_async_copy(
                rows_v.at[nb], acc_sh.at[eb.at[pe0 + 1]], semw).wait()

        @pl.when(j + 1 < NJ)
        def _prefetch():
            pltpu.make_async_copy(
                edata_hbm.at[wid].at[j + 1], eb.at[pl.ds(ne0, 3)], semi).wait()
            pltpu.async_copy(asrc_hbm.at[eb.at[ne0]], avb.at[nb], sem)
            pltpu.async_copy(fc_hbm.at[eb.at[ne0]], rows_v.at[nb], sem)

        @pl.when(j + 2 < NJ)
        def _stage():
            pltpu.async_copy(edata_hbm.at[wid].at[j + 2],
                             eb.at[pl.ds(fe0, 3)], semi)

        for k in range(8):
            o = k * 16
            ae = plsc.bitcast(eb[e0 + 2, pl.ds(o, 16)], jnp.float32)
            exb[b, pl.ds(o, 16)] = jnp.exp(avb[b, pl.ds(o, 16)] + ae)

        def scale_body(r4, c2):
            for u in range(4):
                r = r4 * 4 + u
                a = plsc.load_gather(exb.at[b], [jnp.full((16,), r, jnp.int32)])
                for k in range(8):
                    rows_v[b, r, pl.ds(k * 16, 16)] = (
                        rows_v[b, r, pl.ds(k * 16, 16)] * a)
            return c2
        lax.fori_loop(0, CHUNK // 4, scale_body, 0)

        pltpu.async_copy(exb.at[b], s_sh.at[eb.at[e0 + 1]], semw, add=True)
        pltpu.async_copy(rows_v.at[b], acc_sh.at[eb.at[e0 + 1]], semw, add=True)
        return c
    lax.fori_loop(0, NJ, edge_body, 0)

    bl = (NJ - 1) % 2
    ble = ((NJ - 1) % 4) * 3
    pltpu.make_async_copy(exb.at[bl], s_sh.at[eb.at[ble + 1]], semw).wait()
    pltpu.make_async_copy(rows_v.at[bl], acc_sh.at[eb.at[ble + 1]], semw).wait()

    plsc.subcore_barrier()

    for b in range(STRIPE // CHUNK):
        pltpu.sync_copy(acc_sh.at[pl.ds(base + b * CHUNK, CHUNK)], rows_v.at[0])
        pltpu.sync_copy(rows_v.at[0], part_hbm.at[cid].at[pl.ds(base + b * CHUNK, CHUNK)])
    pltpu.sync_copy(s_sh.at[pl.ds(base, STRIPE)], zero1_v)
    pltpu.sync_copy(zero1_v, s_hbm.at[cid].at[pl.ds(base, STRIPE)])


def _epilogue_body(p_ref, s0_ref, s1_ref, fcd_ref, out_ref):
    p = p_ref[...]
    s = s0_ref[...] + s1_ref[...]
    r = 1.0 / (s + 1e-16)
    out_ref[...] = (p[0] + p[1]) * r + fcd_ref[...]


@jax.jit
def kernel(feat_src, edge_index, feat_edge, W_src, W_dst, b_dst, W_attn_src, W_attn_edge):
    src = edge_index[0]
    dst = edge_index[1]

    was_p = jnp.pad(W_attn_src, ((0, 0), (0, 7)))
    waet_p = jnp.pad(W_attn_edge.T, ((0, 7), (0, 0)))
    b2 = b_dst.reshape(1, D)
    fe_t = feat_edge.T

    g = 25
    bn = N // g
    be = E // g
    fc_src, fc_dst, asrc8, ae8t = pl.pallas_call(
        _matmul_body,
        grid=(g,),
        in_specs=[
            pl.BlockSpec((bn, D), lambda i: (i, 0)),
            pl.BlockSpec((D_EDGE, be), lambda i: (0, i)),
            pl.BlockSpec((D, D), lambda i: (0, 0)),
            pl.BlockSpec((D, D), lambda i: (0, 0)),
            pl.BlockSpec((1, D), lambda i: (0, 0)),
            pl.BlockSpec((D, 8), lambda i: (0, 0)),
            pl.BlockSpec((8, D_EDGE), lambda i: (0, 0)),
        ],
        out_specs=[
            pl.BlockSpec((bn, D), lambda i: (i, 0)),
            pl.BlockSpec((bn, D), lambda i: (i, 0)),
            pl.BlockSpec((bn, 8), lambda i: (i, 0)),
            pl.BlockSpec((8, be), lambda i: (0, i)),
        ],
        out_shape=[
            jax.ShapeDtypeStruct((N, D), jnp.float32),
            jax.ShapeDtypeStruct((N, D), jnp.float32),
            jax.ShapeDtypeStruct((N, 8), jnp.float32),
            jax.ShapeDtypeStruct((8, E), jnp.float32),
        ],
    )(feat_src, fe_t, W_src, W_dst, b2, was_p, waet_p)

    asrc = asrc8[:, 0]
    aedge = ae8t[0]

    pad = EPAD - EPT
    src_p = jnp.pad(src.reshape(NW, EPT), ((0, 0), (0, pad))).reshape(NW, NJ, CHUNK)
    dst_p = jnp.pad(dst.reshape(NW, EPT), ((0, 0), (0, pad)),
                    constant_values=N).reshape(NW, NJ, CHUNK)
    ae_p = jnp.pad(aedge.reshape(NW, EPT), ((0, 0), (0, pad)),
                   constant_values=-1e30).reshape(NW, NJ, CHUNK)
    edata = jnp.stack(
        [src_p, dst_p, lax.bitcast_convert_type(ae_p, jnp.int32)], axis=2)

    sc_fn = pl.kernel(
        _sc_body,
        out_type=(
            jax.ShapeDtypeStruct((NC, NPAD, D), jnp.float32),
            jax.ShapeDtypeStruct((NC, NPAD), jnp.float32),
        ),
        mesh=plsc.VectorSubcoreMesh(core_axis_name="c", subcore_axis_name="s"),
        compiler_params=pltpu.CompilerParams(needs_layout_passes=False),
        scratch_types=[
            pltpu.VMEM((12, CHUNK), jnp.int32),
            pltpu.VMEM((2, CHUNK), jnp.float32),
            pltpu.VMEM((2, CHUNK), jnp.float32),
            pltpu.VMEM((2, CHUNK, D), jnp.float32),
            pltpu.VMEM((STRIPE,), jnp.float32),
            pltpu.VMEM_SHARED((NPAD, D), jnp.float32),
            pltpu.VMEM_SHARED((NPAD,), jnp.float32),
            pltpu.SemaphoreType.DMA,
            pltpu.SemaphoreType.DMA,
            pltpu.SemaphoreType.DMA,
        ],
    )
    part, s_part = sc_fn(asrc, edata, fc_src)

    ge = 10
    bo = N // ge
    out = pl.pallas_call(
        _epilogue_body,
        grid=(ge,),
        in_specs=[
            pl.BlockSpec((NC, bo, D), lambda i: (0, i, 0)),
            pl.BlockSpec((bo, 1), lambda i: (i, 0)),
            pl.BlockSpec((bo, 1), lambda i: (i, 0)),
            pl.BlockSpec((bo, D), lambda i: (i, 0)),
        ],
        out_specs=pl.BlockSpec((bo, D), lambda i: (i, 0)),
        out_shape=jax.ShapeDtypeStruct((N, D), jnp.float32),
    )(part,
      s_part[0, :N].reshape(N, 1), s_part[1, :N].reshape(N, 1), fc_dst)

    return out.reshape(N, 1, D)


# speedup vs baseline: 36.9717x; 1.6738x over previous
import functools

import jax
import jax.numpy as jnp
from jax import lax
from jax.experimental import pallas as pl
from jax.experimental.pallas import tpu as pltpu
from jax.experimental.pallas import tpu_sc as plsc

N = 10000
E = 320000
D = 128
D_EDGE = 16

NC = 2
NS = 16
NW = NC * NS
EPT = E // NW
CHUNK = 128
NBT = E // CHUNK
NJ = 79
NJ_LAST = NBT - (NW - 1) * NJ
STRIPE = 640
NPAD = NS * STRIPE


def _matmul_body(x_ref, fet_ref, ws_ref, wd_ref, b_ref, was_ref, waet_ref,
                 fcs_ref, fcd_ref, asrc_ref, aet_ref):
    x = x_ref[...]
    fcs_ref[...] = jnp.dot(x, ws_ref[...], preferred_element_type=jnp.float32)
    fcd_ref[...] = jnp.dot(x, wd_ref[...], preferred_element_type=jnp.float32) + b_ref[...]
    asrc_ref[...] = jnp.dot(x, was_ref[...], preferred_element_type=jnp.float32)
    ae = jnp.dot(waet_ref[...], fet_ref[...], preferred_element_type=jnp.float32)
    aet_ref[...] = ae[0:1, :]


def _sc_body(asrc_hbm, edata_hbm, fc_hbm,
             part_hbm, s_hbm,
             eb, avidx, avb, exb, rows_v, zero1_v,
             acc_sh, s_sh, sem, semw, semi):
    cid = lax.axis_index("c")
    sid = lax.axis_index("s")
    wid = cid * NS + sid
    ebase = wid * NJ
    njw = jnp.where(wid == NW - 1, NJ_LAST, NJ)
    base = pl.multiple_of(sid * STRIPE, STRIPE)

    z16 = jnp.zeros((16,), jnp.float32)

    def zrows(i, c):
        rows_v[0, i // 8, pl.ds((i % 8) * 16, 16)] = z16
        return c
    lax.fori_loop(0, CHUNK * 8, zrows, 0)

    def z1(i, c):
        zero1_v[pl.ds(i * 16, 16)] = z16
        return c
    lax.fori_loop(0, STRIPE // 16, z1, 0)

    for b in range(STRIPE // CHUNK):
        pltpu.sync_copy(rows_v.at[0], acc_sh.at[pl.ds(base + b * CHUNK, CHUNK)])
    pltpu.sync_copy(zero1_v, s_sh.at[pl.ds(base, STRIPE)])

    plsc.subcore_barrier()

    pltpu.sync_copy(edata_hbm.at[ebase], eb.at[pl.ds(0, 3)])
    for k in range(8):
        o = k * 16
        avidx[0, pl.ds(o, 16)] = eb[0, pl.ds(o, 16)] * 8
    pltpu.async_copy(asrc_hbm.at[avidx.at[0]], avb.at[0], sem)
    pltpu.async_copy(fc_hbm.at[eb.at[0]], rows_v.at[0], sem)
    pltpu.async_copy(edata_hbm.at[ebase + 1], eb.at[pl.ds(3, 3)], semi)

    def edge_body(j, c):
        b = lax.rem(j, 2)
        nb = 1 - b
        e0 = lax.rem(j, 4) * 3
        pe0 = lax.rem(j + 3, 4) * 3
        ne0 = lax.rem(j + 1, 4) * 3
        fe0 = lax.rem(j + 2, 4) * 3
        pltpu.make_async_copy(asrc_hbm.at[avidx.at[b]], avb.at[b], sem).wait()
        pltpu.make_async_copy(fc_hbm.at[eb.at[e0]], rows_v.at[b], sem).wait()

        @pl.when(j >= 1)
        def _drain():
            pltpu.make_async_copy(
                exb.at[nb], s_sh.at[eb.at[pe0 + 1]], semw).wait()
            pltpu.make_async_copy(
                rows_v.at[nb], acc_sh.at[eb.at[pe0 + 1]], semw).wait()

        @pl.when(j + 1 < njw)
        def _prefetch():
            pltpu.make_async_copy(
                edata_hbm.at[ebase + j + 1], eb.at[pl.ds(ne0, 3)], semi).wait()
            for k in range(8):
                o = k * 16
                avidx[nb, pl.ds(o, 16)] = eb[ne0, pl.ds(o, 16)] * 8
            pltpu.async_copy(asrc_hbm.at[avidx.at[nb]], avb.at[nb], sem)
            pltpu.async_copy(fc_hbm.at[eb.at[ne0]], rows_v.at[nb], sem)

        @pl.when(j + 2 < njw)
        def _stage():
            pltpu.async_copy(edata_hbm.at[ebase + j + 2],
                             eb.at[pl.ds(fe0, 3)], semi)

        for k in range(8):
            o = k * 16
            ae = plsc.bitcast(eb[e0 + 2, pl.ds(o, 16)], jnp.float32)
            exb[b, pl.ds(o, 16)] = jnp.exp(avb[b, pl.ds(o, 16)] + ae)

        def scale_body(r4, c2):
            for u in range(4):
                r = r4 * 4 + u
                a = plsc.load_gather(exb.at[b], [jnp.full((16,), r, jnp.int32)])
                for k in range(8):
                    rows_v[b, r, pl.ds(k * 16, 16)] = (
                        rows_v[b, r, pl.ds(k * 16, 16)] * a)
            return c2
        lax.fori_loop(0, CHUNK // 4, scale_body, 0)

        pltpu.async_copy(exb.at[b], s_sh.at[eb.at[e0 + 1]], semw, add=True)
        pltpu.async_copy(rows_v.at[b], acc_sh.at[eb.at[e0 + 1]], semw, add=True)
        return c
    lax.fori_loop(0, njw, edge_body, 0)

    bl = lax.rem(njw - 1, 2)
    ble = lax.rem(njw - 1, 4) * 3
    pltpu.make_async_copy(exb.at[bl], s_sh.at[eb.at[ble + 1]], semw).wait()
    pltpu.make_async_copy(rows_v.at[bl], acc_sh.at[eb.at[ble + 1]], semw).wait()

    plsc.subcore_barrier()

    for b in range(STRIPE // CHUNK):
        pltpu.sync_copy(acc_sh.at[pl.ds(base + b * CHUNK, CHUNK)], rows_v.at[0])
        pltpu.sync_copy(rows_v.at[0], part_hbm.at[cid].at[pl.ds(base + b * CHUNK, CHUNK)])
    pltpu.sync_copy(s_sh.at[pl.ds(base, STRIPE)], zero1_v)
    pltpu.sync_copy(zero1_v, s_hbm.at[cid].at[pl.ds(base, STRIPE)])


def _epilogue_body(p_ref, s0_ref, s1_ref, fcd_ref, out_ref):
    p = p_ref[...]
    s = s0_ref[...] + s1_ref[...]
    r = 1.0 / (s + 1e-16)
    out_ref[...] = (p[0] + p[1]) * r + fcd_ref[...]


@jax.jit
def kernel(feat_src, edge_index, feat_edge, W_src, W_dst, b_dst, W_attn_src, W_attn_edge):
    src = edge_index[0]
    dst = edge_index[1]

    was_p = jnp.pad(W_attn_src, ((0, 0), (0, 7)))
    waet_p = jnp.pad(W_attn_edge.T, ((0, 7), (0, 0)))
    b2 = b_dst.reshape(1, D)
    fe_t = feat_edge.T

    g = 25
    bn = N // g
    be = E // g
    fc_src, fc_dst, asrc8, aet = pl.pallas_call(
        _matmul_body,
        grid=(g,),
        in_specs=[
            pl.BlockSpec((bn, D), lambda i: (i, 0)),
            pl.BlockSpec((D_EDGE, be), lambda i: (0, i)),
            pl.BlockSpec((D, D), lambda i: (0, 0)),
            pl.BlockSpec((D, D), lambda i: (0, 0)),
            pl.BlockSpec((1, D), lambda i: (0, 0)),
            pl.BlockSpec((D, 8), lambda i: (0, 0)),
            pl.BlockSpec((8, D_EDGE), lambda i: (0, 0)),
        ],
        out_specs=[
            pl.BlockSpec((bn, D), lambda i: (i, 0)),
            pl.BlockSpec((bn, D), lambda i: (i, 0)),
            pl.BlockSpec((bn, 8), lambda i: (i, 0)),
            pl.BlockSpec((1, be), lambda i: (0, i)),
        ],
        out_shape=[
            jax.ShapeDtypeStruct((N, D), jnp.float32),
            jax.ShapeDtypeStruct((N, D), jnp.float32),
            jax.ShapeDtypeStruct((N, 8), jnp.float32),
            jax.ShapeDtypeStruct((1, E), jnp.float32),
        ],
    )(feat_src, fe_t, W_src, W_dst, b2, was_p, waet_p)

    asrc = asrc8.reshape(N * 8)
    aedge = aet[0]

    edata = jnp.stack(
        [src.reshape(NBT, CHUNK), dst.reshape(NBT, CHUNK),
         lax.bitcast_convert_type(aedge, jnp.int32).reshape(NBT, CHUNK)],
        axis=1)

    sc_fn = pl.kernel(
        _sc_body,
        out_type=(
            jax.ShapeDtypeStruct((NC, NPAD, D), jnp.float32),
            jax.ShapeDtypeStruct((NC, NPAD), jnp.float32),
        ),
        mesh=plsc.VectorSubcoreMesh(core_axis_name="c", subcore_axis_name="s"),
        compiler_params=pltpu.CompilerParams(needs_layout_passes=False),
        scratch_types=[
            pltpu.VMEM((12, CHUNK), jnp.int32),
            pltpu.VMEM((2, CHUNK), jnp.int32),
            pltpu.VMEM((2, CHUNK), jnp.float32),
            pltpu.VMEM((2, CHUNK), jnp.float32),
            pltpu.VMEM((2, CHUNK, D), jnp.float32),
            pltpu.VMEM((STRIPE,), jnp.float32),
            pltpu.VMEM_SHARED((NPAD, D), jnp.float32),
            pltpu.VMEM_SHARED((NPAD,), jnp.float32),
            pltpu.SemaphoreType.DMA,
            pltpu.SemaphoreType.DMA,
            pltpu.SemaphoreType.DMA,
        ],
    )
    part, s_part = sc_fn(asrc, edata, fc_src)

    ge = 10
    bo = N // ge
    out = pl.pallas_call(
        _epilogue_body,
        grid=(ge,),
        in_specs=[
            pl.BlockSpec((NC, bo, D), lambda i: (0, i, 0)),
            pl.BlockSpec((bo, 1), lambda i: (i, 0)),
            pl.BlockSpec((bo, 1), lambda i: (i, 0)),
            pl.BlockSpec((bo, D), lambda i: (i, 0)),
        ],
        out_specs=pl.BlockSpec((bo, D), lambda i: (i, 0)),
        out_shape=jax.ShapeDtypeStruct((N, D), jnp.float32),
    )(part,
      s_part[0, :N].reshape(N, 1), s_part[1, :N].reshape(N, 1), fc_dst)

    return out.reshape(N, 1, D)


# speedup vs baseline: 36.9925x; 1.0006x over previous
import functools

import jax
import jax.numpy as jnp
from jax import lax
from jax.experimental import pallas as pl
from jax.experimental.pallas import tpu as pltpu
from jax.experimental.pallas import tpu_sc as plsc

N = 10000
E = 320000
D = 128
D_EDGE = 16

NC = 2
NS = 16
NW = NC * NS
EPT = E // NW
CHUNK = 128
NBT = E // CHUNK
NJ = 79
NJ_LAST = NBT - (NW - 1) * NJ
STRIPE = 640
NPAD = NS * STRIPE


def _matmul_body(x_ref, fet_ref, ws_ref, wd_ref, b_ref, was_ref, waet_ref,
                 fcs_ref, fcd_ref, asrc_ref, aet_ref):
    x = x_ref[...]
    fcs_ref[...] = jnp.dot(x, ws_ref[...], preferred_element_type=jnp.float32)
    fcd_ref[...] = jnp.dot(x, wd_ref[...], preferred_element_type=jnp.float32) + b_ref[...]
    asrc_ref[...] = jnp.dot(x, was_ref[...], preferred_element_type=jnp.float32)
    ae = jnp.dot(waet_ref[...], fet_ref[...], preferred_element_type=jnp.float32)
    aet_ref[...] = ae[0:1, :]


def _sc_body(asrc_hbm, edata_hbm, fc_hbm,
             part_hbm, s_hbm,
             eb, avidx, avb, exb, rows_v, zero1_v,
             acc_sh, s_sh, sem, semw, semi):
    cid = lax.axis_index("c")
    sid = lax.axis_index("s")
    wid = cid * NS + sid
    ebase = wid * NJ
    njw = jnp.where(wid == NW - 1, NJ_LAST, NJ)
    base = pl.multiple_of(sid * STRIPE, STRIPE)

    z16 = jnp.zeros((16,), jnp.float32)

    def zrows(i, c):
        rows_v[0, i // 8, pl.ds((i % 8) * 16, 16)] = z16
        return c
    lax.fori_loop(0, CHUNK * 8, zrows, 0)

    def z1(i, c):
        zero1_v[pl.ds(i * 16, 16)] = z16
        return c
    lax.fori_loop(0, STRIPE // 16, z1, 0)

    for b in range(STRIPE // CHUNK):
        pltpu.sync_copy(rows_v.at[0], acc_sh.at[pl.ds(base + b * CHUNK, CHUNK)])
    pltpu.sync_copy(zero1_v, s_sh.at[pl.ds(base, STRIPE)])

    plsc.subcore_barrier()

    pltpu.sync_copy(edata_hbm.at[ebase], eb.at[pl.ds(0, 3)])
    for k in range(8):
        o = k * 16
        avidx[0, pl.ds(o, 16)] = eb[0, pl.ds(o, 16)] * 8
    pltpu.async_copy(asrc_hbm.at[avidx.at[0]], avb.at[0], sem)
    pltpu.async_copy(fc_hbm.at[eb.at[0]], rows_v.at[0], sem)
    pltpu.async_copy(edata_hbm.at[ebase + 1], eb.at[pl.ds(3, 3)], semi)

    def _ex_scale(b, e0):
        for k in range(8):
            o = k * 16
            ae = plsc.bitcast(eb[e0 + 2, pl.ds(o, 16)], jnp.float32)
            exb[b, pl.ds(o, 16)] = jnp.exp(avb[b, pl.ds(o, 16)] + ae)

        def scale_body(r8, c2):
            for u in range(8):
                r = r8 * 8 + u
                a = plsc.load_gather(exb.at[b], [jnp.full((16,), r, jnp.int32)])
                for k in range(8):
                    rows_v[b, r, pl.ds(k * 16, 16)] = (
                        rows_v[b, r, pl.ds(k * 16, 16)] * a)
            return c2
        lax.fori_loop(0, CHUNK // 8, scale_body, 0)

    def _wait_gathers(b, e0):
        pltpu.make_async_copy(asrc_hbm.at[avidx.at[b]], avb.at[b], sem).wait()
        pltpu.make_async_copy(fc_hbm.at[eb.at[e0]], rows_v.at[b], sem).wait()

    def _drain_scatters(b, e0):
        pltpu.make_async_copy(exb.at[b], s_sh.at[eb.at[e0 + 1]], semw).wait()
        pltpu.make_async_copy(rows_v.at[b], acc_sh.at[eb.at[e0 + 1]], semw).wait()

    def _launch_gathers(b, e0):
        for k in range(8):
            o = k * 16
            avidx[b, pl.ds(o, 16)] = eb[e0, pl.ds(o, 16)] * 8
        pltpu.async_copy(asrc_hbm.at[avidx.at[b]], avb.at[b], sem)
        pltpu.async_copy(fc_hbm.at[eb.at[e0]], rows_v.at[b], sem)

    def _issue_scatters(b, e0):
        pltpu.async_copy(exb.at[b], s_sh.at[eb.at[e0 + 1]], semw, add=True)
        pltpu.async_copy(rows_v.at[b], acc_sh.at[eb.at[e0 + 1]], semw, add=True)

    def pair_body(t, c):
        p6 = lax.rem(t, 2) * 6
        e00 = p6
        e01 = p6 + 3
        q6 = 6 - p6
        pe0 = 9 - p6

        _wait_gathers(0, e00)

        @pl.when(t >= 1)
        def _():
            _drain_scatters(1, pe0)
        pltpu.make_async_copy(
            edata_hbm.at[ebase + 2 * t + 1], eb.at[pl.ds(e01, 3)], semi).wait()
        _launch_gathers(1, e01)
        pltpu.async_copy(edata_hbm.at[ebase + 2 * t + 2],
                         eb.at[pl.ds(q6, 3)], semi)
        _ex_scale(0, e00)
        _issue_scatters(0, e00)

        _wait_gathers(1, e01)
        _drain_scatters(0, e00)
        pltpu.make_async_copy(
            edata_hbm.at[ebase + 2 * t + 2], eb.at[pl.ds(q6, 3)], semi).wait()
        _launch_gathers(0, q6)

        @pl.when(2 * t + 3 < njw)
        def _():
            pltpu.async_copy(edata_hbm.at[ebase + 2 * t + 3],
                             eb.at[pl.ds(q6 + 3, 3)], semi)
        _ex_scale(1, e01)
        _issue_scatters(1, e01)
        return c
    lax.fori_loop(0, (njw - 1) // 2, pair_body, 0)

    _wait_gathers(0, 6)
    _drain_scatters(1, 3)
    _ex_scale(0, 6)
    _issue_scatters(0, 6)
    _drain_scatters(0, 6)

    plsc.subcore_barrier()

    for b in range(STRIPE // CHUNK):
        pltpu.sync_copy(acc_sh.at[pl.ds(base + b * CHUNK, CHUNK)], rows_v.at[0])
        pltpu.sync_copy(rows_v.at[0], part_hbm.at[cid].at[pl.ds(base + b * CHUNK, CHUNK)])
    pltpu.sync_copy(s_sh.at[pl.ds(base, STRIPE)], zero1_v)
    pltpu.sync_copy(zero1_v, s_hbm.at[cid].at[pl.ds(base, STRIPE)])


def _epilogue_body(p_ref, s0_ref, s1_ref, fcd_ref, out_ref):
    p = p_ref[...]
    s = s0_ref[...] + s1_ref[...]
    r = 1.0 / (s + 1e-16)
    out_ref[...] = (p[0] + p[1]) * r + fcd_ref[...]


@jax.jit
def kernel(feat_src, edge_index, feat_edge, W_src, W_dst, b_dst, W_attn_src, W_attn_edge):
    src = edge_index[0]
    dst = edge_index[1]

    was_p = jnp.pad(W_attn_src, ((0, 0), (0, 7)))
    waet_p = jnp.pad(W_attn_edge.T, ((0, 7), (0, 0)))
    b2 = b_dst.reshape(1, D)
    fe_t = feat_edge.T

    g = 25
    bn = N // g
    be = E // g
    fc_src, fc_dst, asrc8, aet = pl.pallas_call(
        _matmul_body,
        grid=(g,),
        in_specs=[
            pl.BlockSpec((bn, D), lambda i: (i, 0)),
            pl.BlockSpec((D_EDGE, be), lambda i: (0, i)),
            pl.BlockSpec((D, D), lambda i: (0, 0)),
            pl.BlockSpec((D, D), lambda i: (0, 0)),
            pl.BlockSpec((1, D), lambda i: (0, 0)),
            pl.BlockSpec((D, 8), lambda i: (0, 0)),
            pl.BlockSpec((8, D_EDGE), lambda i: (0, 0)),
        ],
        out_specs=[
            pl.BlockSpec((bn, D), lambda i: (i, 0)),
            pl.BlockSpec((bn, D), lambda i: (i, 0)),
            pl.BlockSpec((bn, 8), lambda i: (i, 0)),
            pl.BlockSpec((1, be), lambda i: (0, i)),
        ],
        out_shape=[
            jax.ShapeDtypeStruct((N, D), jnp.float32),
            jax.ShapeDtypeStruct((N, D), jnp.float32),
            jax.ShapeDtypeStruct((N, 8), jnp.float32),
            jax.ShapeDtypeStruct((1, E), jnp.float32),
        ],
    )(feat_src, fe_t, W_src, W_dst, b2, was_p, waet_p)

    asrc = asrc8.reshape(N * 8)
    aedge = aet[0]

    edata = jnp.stack(
        [src.reshape(NBT, CHUNK), dst.reshape(NBT, CHUNK),
         lax.bitcast_convert_type(aedge, jnp.int32).reshape(NBT, CHUNK)],
        axis=1)

    sc_fn = pl.kernel(
        _sc_body,
        out_type=(
            jax.ShapeDtypeStruct((NC, NPAD, D), jnp.float32),
            jax.ShapeDtypeStruct((NC, NPAD), jnp.float32),
        ),
        mesh=plsc.VectorSubcoreMesh(core_axis_name="c", subcore_axis_name="s"),
        compiler_params=pltpu.CompilerParams(needs_layout_passes=False),
        scratch_types=[
            pltpu.VMEM((12, CHUNK), jnp.int32),
            pltpu.VMEM((2, CHUNK), jnp.int32),
            pltpu.VMEM((2, CHUNK), jnp.float32),
            pltpu.VMEM((2, CHUNK), jnp.float32),
            pltpu.VMEM((2, CHUNK, D), jnp.float32),
            pltpu.VMEM((STRIPE,), jnp.float32),
            pltpu.VMEM_SHARED((NPAD, D), jnp.float32),
            pltpu.VMEM_SHARED((NPAD,), jnp.float32),
            pltpu.SemaphoreType.DMA,
            pltpu.SemaphoreType.DMA,
            pltpu.SemaphoreType.DMA,
        ],
    )
    part, s_part = sc_fn(asrc, edata, fc_src)

    ge = 10
    bo = N // ge
    out = pl.pallas_call(
        _epilogue_body,
        grid=(ge,),
        in_specs=[
            pl.BlockSpec((NC, bo, D), lambda i: (0, i, 0)),
            pl.BlockSpec((bo, 1), lambda i: (i, 0)),
            pl.BlockSpec((bo, 1), lambda i: (i, 0)),
            pl.BlockSpec((bo, D), lambda i: (i, 0)),
        ],
        out_specs=pl.BlockSpec((bo, D), lambda i: (i, 0)),
        out_shape=jax.ShapeDtypeStruct((N, D), jnp.float32),
    )(part,
      s_part[0, :N].reshape(N, 1), s_part[1, :N].reshape(N, 1), fc_dst)

    return out.reshape(N, 1, D)


# speedup vs baseline: 37.8682x; 1.0237x over previous
import functools

import jax
import jax.numpy as jnp
from jax import lax
from jax.experimental import pallas as pl
from jax.experimental.pallas import tpu as pltpu
from jax.experimental.pallas import tpu_sc as plsc

N = 10000
E = 320000
D = 128
D_EDGE = 16

NC = 2
NS = 16
NW = NC * NS
EPT = E // NW
CHUNK = 128
NBT = E // CHUNK
NJ = 79
NJ_LAST = NBT - (NW - 1) * NJ
STRIPE = 640
NPAD = NS * STRIPE


def _matmul_body(x_ref, fet_ref, ws_ref, wd_ref, b_ref, was_ref, waet_ref,
                 fcs_ref, fcd_ref, asrc_ref, aet_ref):
    x = x_ref[...]
    fcs_ref[...] = jnp.dot(x, ws_ref[...], preferred_element_type=jnp.float32)
    fcd_ref[...] = jnp.dot(x, wd_ref[...], preferred_element_type=jnp.float32) + b_ref[...]
    asrc_ref[...] = jnp.dot(x, was_ref[...], preferred_element_type=jnp.float32)
    ae = jnp.dot(waet_ref[...], fet_ref[...], preferred_element_type=jnp.float32)
    aet_ref[...] = ae[0:1, :]


def _sc_body(asrc_hbm, ei_hbm, ae_hbm, fc_hbm,
             part_hbm, s_hbm,
             eb, aeb, avidx, avb, exb, rows_v, zero1_v,
             acc_sh, s_sh, sem, semw, semi):
    cid = lax.axis_index("c")
    sid = lax.axis_index("s")
    wid = cid * NS + sid
    ebase = wid * NJ
    njw = jnp.where(wid == NW - 1, NJ_LAST, NJ)
    base = pl.multiple_of(sid * STRIPE, STRIPE)

    z16 = jnp.zeros((16,), jnp.float32)

    def zrows(i, c):
        rows_v[0, i // 8, pl.ds((i % 8) * 16, 16)] = z16
        return c
    lax.fori_loop(0, CHUNK * 8, zrows, 0)

    def z1(i, c):
        zero1_v[pl.ds(i * 16, 16)] = z16
        return c
    lax.fori_loop(0, STRIPE // 16, z1, 0)

    for b in range(STRIPE // CHUNK):
        pltpu.sync_copy(rows_v.at[0], acc_sh.at[pl.ds(base + b * CHUNK, CHUNK)])
    pltpu.sync_copy(zero1_v, s_sh.at[pl.ds(base, STRIPE)])

    plsc.subcore_barrier()

    def _stage(m, g):
        off = g * CHUNK
        pltpu.async_copy(ei_hbm.at[0].at[pl.ds(off, CHUNK)], eb.at[2 * m], semi)
        pltpu.async_copy(ei_hbm.at[1].at[pl.ds(off, CHUNK)], eb.at[2 * m + 1], semi)
        pltpu.async_copy(ae_hbm.at[0].at[pl.ds(off, CHUNK)], aeb.at[m], semi)

    def _wait_stage(m, g):
        off = g * CHUNK
        pltpu.make_async_copy(
            ei_hbm.at[0].at[pl.ds(off, CHUNK)], eb.at[2 * m], semi).wait()
        pltpu.make_async_copy(
            ei_hbm.at[1].at[pl.ds(off, CHUNK)], eb.at[2 * m + 1], semi).wait()
        pltpu.make_async_copy(
            ae_hbm.at[0].at[pl.ds(off, CHUNK)], aeb.at[m], semi).wait()

    _stage(0, ebase)
    _wait_stage(0, ebase)
    for k in range(8):
        o = k * 16
        avidx[0, pl.ds(o, 16)] = eb[0, pl.ds(o, 16)] * 8
    pltpu.async_copy(asrc_hbm.at[avidx.at[0]], avb.at[0], sem)
    pltpu.async_copy(fc_hbm.at[eb.at[0]], rows_v.at[0], sem)
    _stage(1, ebase + 1)

    def _ex_scale(b, m):
        for k in range(8):
            o = k * 16
            exb[b, pl.ds(o, 16)] = jnp.exp(avb[b, pl.ds(o, 16)] + aeb[m, pl.ds(o, 16)])

        def scale_body(r8, c2):
            for u in range(8):
                r = r8 * 8 + u
                a = plsc.load_gather(exb.at[b], [jnp.full((16,), r, jnp.int32)])
                for k in range(8):
                    rows_v[b, r, pl.ds(k * 16, 16)] = (
                        rows_v[b, r, pl.ds(k * 16, 16)] * a)
            return c2
        lax.fori_loop(0, CHUNK // 8, scale_body, 0)

    def _wait_gathers(b, m):
        pltpu.make_async_copy(asrc_hbm.at[avidx.at[b]], avb.at[b], sem).wait()
        pltpu.make_async_copy(fc_hbm.at[eb.at[2 * m]], rows_v.at[b], sem).wait()

    def _drain_scatters(b, m):
        pltpu.make_async_copy(exb.at[b], s_sh.at[eb.at[2 * m + 1]], semw).wait()
        pltpu.make_async_copy(
            rows_v.at[b], acc_sh.at[eb.at[2 * m + 1]], semw).wait()

    def _launch_gathers(b, m):
        for k in range(8):
            o = k * 16
            avidx[b, pl.ds(o, 16)] = eb[2 * m, pl.ds(o, 16)] * 8
        pltpu.async_copy(asrc_hbm.at[avidx.at[b]], avb.at[b], sem)
        pltpu.async_copy(fc_hbm.at[eb.at[2 * m]], rows_v.at[b], sem)

    def _issue_scatters(b, m):
        pltpu.async_copy(exb.at[b], s_sh.at[eb.at[2 * m + 1]], semw, add=True)
        pltpu.async_copy(
            rows_v.at[b], acc_sh.at[eb.at[2 * m + 1]], semw, add=True)

    def pair_body(t, c):
        p2 = lax.rem(t, 2) * 2
        m0 = p2
        m1 = p2 + 1
        q = 2 - p2
        pm = 3 - p2

        _wait_gathers(0, m0)

        @pl.when(t >= 1)
        def _():
            _drain_scatters(1, pm)
        _wait_stage(m1, ebase + 2 * t + 1)
        _launch_gathers(1, m1)
        _stage(q, ebase + 2 * t + 2)
        _ex_scale(0, m0)
        _issue_scatters(0, m0)

        _wait_gathers(1, m1)
        _drain_scatters(0, m0)
        _wait_stage(q, ebase + 2 * t + 2)
        _launch_gathers(0, q)

        @pl.when(2 * t + 3 < njw)
        def _():
            _stage(pm, ebase + 2 * t + 3)
        _ex_scale(1, m1)
        _issue_scatters(1, m1)
        return c
    lax.fori_loop(0, (njw - 1) // 2, pair_body, 0)

    _wait_gathers(0, 2)
    _drain_scatters(1, 1)
    _ex_scale(0, 2)
    _issue_scatters(0, 2)
    _drain_scatters(0, 2)

    plsc.subcore_barrier()

    pltpu.sync_copy(acc_sh.at[pl.ds(base, STRIPE)],
                    part_hbm.at[cid].at[pl.ds(base, STRIPE)])
    pltpu.sync_copy(s_sh.at[pl.ds(base, STRIPE)],
                    s_hbm.at[cid].at[pl.ds(base, STRIPE)])


def _epilogue_body(p_ref, s0_ref, s1_ref, fcd_ref, out_ref):
    p = p_ref[...]
    s = s0_ref[...] + s1_ref[...]
    r = 1.0 / (s + 1e-16)
    out_ref[...] = (p[0] + p[1]) * r + fcd_ref[...]


@jax.jit
def kernel(feat_src, edge_index, feat_edge, W_src, W_dst, b_dst, W_attn_src, W_attn_edge):
    src = edge_index[0]
    dst = edge_index[1]

    was_p = jnp.pad(W_attn_src, ((0, 0), (0, 7)))
    waet_p = jnp.pad(W_attn_edge.T, ((0, 7), (0, 0)))
    b2 = b_dst.reshape(1, D)
    fe_t = feat_edge.T

    g = 25
    bn = N // g
    be = E // g
    fc_src, fc_dst, asrc8, aet = pl.pallas_call(
        _matmul_body,
        grid=(g,),
        in_specs=[
            pl.BlockSpec((bn, D), lambda i: (i, 0)),
            pl.BlockSpec((D_EDGE, be), lambda i: (0, i)),
            pl.BlockSpec((D, D), lambda i: (0, 0)),
            pl.BlockSpec((D, D), lambda i: (0, 0)),
            pl.BlockSpec((1, D), lambda i: (0, 0)),
            pl.BlockSpec((D, 8), lambda i: (0, 0)),
            pl.BlockSpec((8, D_EDGE), lambda i: (0, 0)),
        ],
        out_specs=[
            pl.BlockSpec((bn, D), lambda i: (i, 0)),
            pl.BlockSpec((bn, D), lambda i: (i, 0)),
            pl.BlockSpec((bn, 8), lambda i: (i, 0)),
            pl.BlockSpec((1, be), lambda i: (0, i)),
        ],
        out_shape=[
            jax.ShapeDtypeStruct((N, D), jnp.float32),
            jax.ShapeDtypeStruct((N, D), jnp.float32),
            jax.ShapeDtypeStruct((N, 8), jnp.float32),
            jax.ShapeDtypeStruct((1, E), jnp.float32),
        ],
    )(feat_src, fe_t, W_src, W_dst, b2, was_p, waet_p)

    asrc = asrc8.reshape(N * 8)
    aedge = aet[0]


    sc_fn = pl.kernel(
        _sc_body,
        out_type=(
            jax.ShapeDtypeStruct((NC, NPAD, D), jnp.float32),
            jax.ShapeDtypeStruct((NC, NPAD), jnp.float32),
        ),
        mesh=plsc.VectorSubcoreMesh(core_axis_name="c", subcore_axis_name="s"),
        compiler_params=pltpu.CompilerParams(needs_layout_passes=False),
        scratch_types=[
            pltpu.VMEM((8, CHUNK), jnp.int32),
            pltpu.VMEM((4, CHUNK), jnp.float32),
            pltpu.VMEM((2, CHUNK), jnp.int32),
            pltpu.VMEM((2, CHUNK), jnp.float32),
            pltpu.VMEM((2, CHUNK), jnp.float32),
            pltpu.VMEM((2, CHUNK, D), jnp.float32),
            pltpu.VMEM((STRIPE,), jnp.float32),
            pltpu.VMEM_SHARED((NPAD, D), jnp.float32),
            pltpu.VMEM_SHARED((NPAD,), jnp.float32),
            pltpu.SemaphoreType.DMA,
            pltpu.SemaphoreType.DMA,
            pltpu.SemaphoreType.DMA,
        ],
    )
    part, s_part = sc_fn(asrc, edge_index, aet, fc_src)

    ge = 10
    bo = N // ge
    out = pl.pallas_call(
        _epilogue_body,
        grid=(ge,),
        in_specs=[
            pl.BlockSpec((NC, bo, D), lambda i: (0, i, 0)),
            pl.BlockSpec((bo, 1), lambda i: (i, 0)),
            pl.BlockSpec((bo, 1), lambda i: (i, 0)),
            pl.BlockSpec((bo, D), lambda i: (i, 0)),
        ],
        out_specs=pl.BlockSpec((bo, D), lambda i: (i, 0)),
        out_shape=jax.ShapeDtypeStruct((N, D), jnp.float32),
    )(part,
      s_part[0, :N].reshape(N, 1), s_part[1, :N].reshape(N, 1), fc_dst)

    return out.reshape(N, 1, D)
